# baseline (speedup 1.0000x reference)
import jax
import jax.numpy as jnp
from jax import lax
from jax.experimental import pallas as pl
from jax.experimental.pallas import tpu as pltpu
from jax.experimental.pallas import tpu_sc as plsc

B, F, V = 16384, 26, 1000000
L = 16
NC, NS = 2, 16
NW = NC * NS
ROWS_W = B // NW
E = ROWS_W * F


def _sc_gather_body(idx_hbm, w_hbm, g_hbm, idx_v, g_v, sem):
    wid = lax.axis_index("s") * NC + lax.axis_index("c")
    base = wid * E
    pltpu.sync_copy(idx_hbm.at[pl.ds(base, E)], idx_v)
    pltpu.async_copy(w_hbm.at[idx_v], g_v, sem).wait()
    pltpu.sync_copy(g_v, g_hbm.at[pl.ds(base, E)])


def _sc_reduce_body(g_hbm, vals_hbm, b_hbm, out_hbm,
                    g_v, v_v, b_v, out_v):
    wid = lax.axis_index("s") * NC + lax.axis_index("c")
    base = wid * E
    pltpu.sync_copy(g_hbm.at[pl.ds(base, E)], g_v)
    pltpu.sync_copy(vals_hbm.at[pl.ds(base, E)], v_v)
    pltpu.sync_copy(b_hbm, b_v)
    bvec = b_v[...]

    def colgroup(sg, carry):
        acc = bvec
        for f in range(F):
            s = pl.ds(f * ROWS_W + sg * L, L)
            acc = acc + g_v[s] * v_v[s]
        out_v[pl.ds(sg * L, L)] = 1.0 / (1.0 + jnp.exp(-acc))
        return carry
    lax.fori_loop(0, ROWS_W // L, colgroup, 0)

    pltpu.sync_copy(out_v, out_hbm.at[pl.ds(wid * ROWS_W, ROWS_W)])


def kernel(feature_idx, feature_vals, W, b):
    idx_bt = (feature_idx.astype(jnp.int32)
              .reshape(NW, ROWS_W, F).transpose(0, 2, 1).reshape(NW * E))
    vals_bt = feature_vals.reshape(NW, ROWS_W, F).transpose(0, 2, 1).reshape(NW * E)
    b16 = jnp.broadcast_to(jnp.asarray(b, jnp.float32).reshape(()), (L,))

    mesh = plsc.VectorSubcoreMesh(core_axis_name="c", subcore_axis_name="s")
    gathered = pl.kernel(
        _sc_gather_body,
        out_type=jax.ShapeDtypeStruct((NW * E,), jnp.float32),
        mesh=mesh,
        scratch_types=[
            pltpu.VMEM((E,), jnp.int32),
            pltpu.VMEM((E,), jnp.float32),
            pltpu.SemaphoreType.DMA,
        ],
    )(idx_bt, W)

    return pl.kernel(
        _sc_reduce_body,
        out_type=jax.ShapeDtypeStruct((B,), jnp.float32),
        mesh=mesh,
        scratch_types=[
            pltpu.VMEM((E,), jnp.float32),
            pltpu.VMEM((E,), jnp.float32),
            pltpu.VMEM((L,), jnp.float32),
            pltpu.VMEM((ROWS_W,), jnp.float32),
        ],
    )(gathered, vals_bt, b16)


# speedup vs baseline: 1.0286x; 1.0286x over previous
import jax
import jax.numpy as jnp
from jax import lax
from jax.experimental import pallas as pl
from jax.experimental.pallas import tpu as pltpu
from jax.experimental.pallas import tpu_sc as plsc

B, F, V = 16384, 26, 1000000
L = 16
NC, NS = 2, 16
NW = NC * NS
ROWS_W = B // NW
E = ROWS_W * F
FSPLIT = 20
SPLIT = FSPLIT * ROWS_W


def _sc_body(idx_hbm, vals_hbm, w_hbm, b_hbm, out_hbm,
             idx_v, v_v, g_v, b_v, acc_v, out_v, sem0, sem1):
    wid = lax.axis_index("s") * NC + lax.axis_index("c")
    base = wid * E

    pltpu.sync_copy(idx_hbm.at[pl.ds(base, E)], idx_v)
    cp0 = pltpu.async_copy(
        w_hbm.at[idx_v.at[pl.ds(0, SPLIT)]], g_v.at[pl.ds(0, SPLIT)], sem0)
    cp1 = pltpu.async_copy(
        w_hbm.at[idx_v.at[pl.ds(SPLIT, E - SPLIT)]],
        g_v.at[pl.ds(SPLIT, E - SPLIT)], sem1)
    pltpu.sync_copy(vals_hbm.at[pl.ds(base, E)], v_v)
    pltpu.sync_copy(b_hbm, b_v)

    bvec = b_v[...]

    cp0.wait()

    def part0(sg, carry):
        acc = bvec
        for f in range(FSPLIT):
            s = pl.ds(f * ROWS_W + sg * L, L)
            acc = acc + g_v[s] * v_v[s]
        acc_v[pl.ds(sg * L, L)] = acc
        return carry
    lax.fori_loop(0, ROWS_W // L, part0, 0)

    cp1.wait()

    def part1(sg, carry):
        acc = acc_v[pl.ds(sg * L, L)]
        for f in range(FSPLIT, F):
            s = pl.ds(f * ROWS_W + sg * L, L)
            acc = acc + g_v[s] * v_v[s]
        out_v[pl.ds(sg * L, L)] = 1.0 / (1.0 + jnp.exp(-acc))
        return carry
    lax.fori_loop(0, ROWS_W // L, part1, 0)

    pltpu.sync_copy(out_v, out_hbm.at[pl.ds(wid * ROWS_W, ROWS_W)])


def kernel(feature_idx, feature_vals, W, b):
    idx_bt = (feature_idx.astype(jnp.int32)
              .reshape(NW, ROWS_W, F).transpose(0, 2, 1).reshape(NW * E))
    vals_bt = feature_vals.reshape(NW, ROWS_W, F).transpose(0, 2, 1).reshape(NW * E)
    b16 = jnp.broadcast_to(jnp.asarray(b, jnp.float32).reshape(()), (L,))

    mesh = plsc.VectorSubcoreMesh(core_axis_name="c", subcore_axis_name="s")
    run = pl.kernel(
        _sc_body,
        out_type=jax.ShapeDtypeStruct((B,), jnp.float32),
        mesh=mesh,
        scratch_types=[
            pltpu.VMEM((E,), jnp.int32),
            pltpu.VMEM((E,), jnp.float32),
            pltpu.VMEM((E,), jnp.float32),
            pltpu.VMEM((L,), jnp.float32),
            pltpu.VMEM((ROWS_W,), jnp.float32),
            pltpu.VMEM((ROWS_W,), jnp.float32),
            pltpu.SemaphoreType.DMA,
            pltpu.SemaphoreType.DMA,
        ],
    )
    return run(idx_bt, vals_bt, W, b16)


# speedup vs baseline: 1.0447x; 1.0156x over previous
import jax
import jax.numpy as jnp
from jax import lax
from jax.experimental import pallas as pl
from jax.experimental.pallas import tpu as pltpu
from jax.experimental.pallas import tpu_sc as plsc

B, F, V = 16384, 26, 1000000
L = 16
NC, NS = 2, 16
NW = NC * NS
ROWS_W = B // NW
E = ROWS_W * F
FSPLIT = 13
SPLIT = FSPLIT * ROWS_W


def _sc_body(idx_hbm, vals_hbm, w_hbm, b_hbm, out_hbm,
             idx_v, v_v, g_v, b_v, acc_v, out_v, sem0, sem1):
    wid = lax.axis_index("s") * NC + lax.axis_index("c")
    base = wid * E

    pltpu.sync_copy(idx_hbm.at[pl.ds(base, E)], idx_v)
    cp0 = pltpu.async_copy(
        w_hbm.at[idx_v.at[pl.ds(0, SPLIT)]], g_v.at[pl.ds(0, SPLIT)], sem0)
    cp1 = pltpu.async_copy(
        w_hbm.at[idx_v.at[pl.ds(SPLIT, E - SPLIT)]],
        g_v.at[pl.ds(SPLIT, E - SPLIT)], sem1)
    pltpu.sync_copy(vals_hbm.at[pl.ds(base, E)], v_v)
    pltpu.sync_copy(b_hbm, b_v)

    bvec = b_v[...]

    cp0.wait()

    def part0(sg, carry):
        acc = bvec
        for f in range(FSPLIT):
            s = pl.ds(f * ROWS_W + sg * L, L)
            acc = acc + g_v[s] * v_v[s]
        acc_v[pl.ds(sg * L, L)] = acc
        return carry
    lax.fori_loop(0, ROWS_W // L, part0, 0)

    cp1.wait()

    def part1(sg, carry):
        acc = acc_v[pl.ds(sg * L, L)]
        for f in range(FSPLIT, F):
            s = pl.ds(f * ROWS_W + sg * L, L)
            acc = acc + g_v[s] * v_v[s]
        out_v[pl.ds(sg * L, L)] = 1.0 / (1.0 + jnp.exp(-acc))
        return carry
    lax.fori_loop(0, ROWS_W // L, part1, 0)

    pltpu.sync_copy(out_v, out_hbm.at[pl.ds(wid * ROWS_W, ROWS_W)])


def kernel(feature_idx, feature_vals, W, b):
    idx_bt = (feature_idx.astype(jnp.int32)
              .reshape(NW, ROWS_W, F).transpose(0, 2, 1).reshape(NW * E))
    vals_bt = feature_vals.reshape(NW, ROWS_W, F).transpose(0, 2, 1).reshape(NW * E)
    b16 = jnp.broadcast_to(jnp.asarray(b, jnp.float32).reshape(()), (L,))

    mesh = plsc.VectorSubcoreMesh(core_axis_name="c", subcore_axis_name="s")
    run = pl.kernel(
        _sc_body,
        out_type=jax.ShapeDtypeStruct((B,), jnp.float32),
        mesh=mesh,
        scratch_types=[
            pltpu.VMEM((E,), jnp.int32),
            pltpu.VMEM((E,), jnp.float32),
            pltpu.VMEM((E,), jnp.float32),
            pltpu.VMEM((L,), jnp.float32),
            pltpu.VMEM((ROWS_W,), jnp.float32),
            pltpu.VMEM((ROWS_W,), jnp.float32),
            pltpu.SemaphoreType.DMA,
            pltpu.SemaphoreType.DMA,
        ],
    )
    return run(idx_bt, vals_bt, W, b16)
